# baseline (speedup 1.0000x reference)
import functools

import jax
import jax.numpy as jnp
import numpy as np
from jax import lax
from jax.experimental import pallas as pl
from jax.experimental.pallas import tpu as pltpu
from jax.experimental.pallas import tpu_sc as plsc

_VOCAB = 1000000
_DEPTH = 64
_LENGTH = 512
_BATCH = 1024

_LANES = 16
_VBLK = 128
_CH = 256


def _pos_encoding_t_np(length, depth):
    pos = np.arange(length)[:, None]
    i = np.arange(depth)[None, :]
    angle_rates = 1.0 / np.power(10000, 2 * (i // 2) / np.float32(depth))
    angle_rads = pos * angle_rates
    angle_rads[:, 0::2] = np.sin(angle_rads[:, 0::2])
    angle_rads[:, 1::2] = np.cos(angle_rads[:, 1::2])
    return np.ascontiguousarray(angle_rads.astype(np.float32).T)


def _make_relayout_kernel():
    info = plsc.get_sparse_core_info()
    nc, ns = info.num_cores, info.num_subcores
    nw = nc * ns
    n_blk = _VOCAB // _VBLK
    tail_v = _VOCAB - n_blk * _VBLK
    mesh = plsc.VectorSubcoreMesh(core_axis_name="c", subcore_axis_name="s")

    @functools.partial(
        pl.kernel,
        out_type=jax.ShapeDtypeStruct((_VOCAB // 2, 2 * _DEPTH), jnp.float32),
        mesh=mesh,
        scratch_types=[
            pltpu.VMEM((2, _DEPTH, _VBLK + 1), jnp.float32),
            pltpu.VMEM((2, _DEPTH, _VBLK), jnp.float32),
            pltpu.VMEM((tail_v // 2, 2 * _DEPTH), jnp.float32),
            pltpu.SemaphoreType.DMA,
            pltpu.SemaphoreType.DMA,
            pltpu.SemaphoreType.DMA,
            pltpu.SemaphoreType.DMA,
        ],
        compiler_params=pltpu.CompilerParams(use_tc_tiling_on_sc=True, needs_layout_passes=False, disable_bounds_checks=True),
    )
    def k(tt_hbm, tail2_hbm, t2_hbm, slab_v, tout_v, tail_v_buf,
          in_sem0, in_sem1, out_sem0, out_sem1):
        wid = lax.axis_index("s") * nc + lax.axis_index("c")
        in_sems = [in_sem0, in_sem1]
        out_sems = [out_sem0, out_sem1]
        iota = lax.iota(jnp.int32, _LANES)
        half_iota = lax.shift_right_logical(iota, 1)
        parity64 = lax.shift_left(lax.bitwise_and(iota, 1), 6)

        n_mine = (n_blk - 1 - wid) // nw + 1

        def vb_of(i):
            return wid + i * nw

        def start_in(i, slot):
            vb = vb_of(i)
            for a in range(_DEPTH // 8):
                pltpu.async_copy(
                    tt_hbm.at[pl.ds(8 * a, 8), pl.ds(vb * _VBLK, _VBLK)],
                    slab_v.at[slot, pl.ds(8 * a, 8), pl.ds(0, _VBLK)],
                    in_sems[slot])

        d16s = [c * _LANES + iota for c in range(4)]

        def transpose_slab(slot):
            def body_q(ql, _):
                for c in range(8):
                    v16 = jnp.full((_LANES,), 2 * ql + c // 4, jnp.int32)
                    vals = plsc.load_gather(slab_v.at[slot], [d16s[c % 4], v16])
                    tout_v[slot, ql, pl.ds(c * _LANES, _LANES)] = vals
                return 0

            lax.fori_loop(0, _VBLK // 2, body_q, 0)

        @pl.when(n_mine > 0)
        def _():
            start_in(0, 0)

        def step(i, slot):
            @pl.when(i + 1 < n_mine)
            def _():
                start_in(i + 1, 1 - slot)

            pltpu.make_async_copy(
                tt_hbm.at[:, pl.ds(0, _VBLK)],
                slab_v.at[slot, :, pl.ds(0, _VBLK)],
                in_sems[slot]).wait()

            @pl.when(i >= 2)
            def _():
                pltpu.make_async_copy(
                    tout_v.at[slot],
                    t2_hbm.at[pl.ds(0, _VBLK // 2)],
                    out_sems[slot]).wait()

            transpose_slab(slot)
            vb = vb_of(i)
            pltpu.async_copy(
                tout_v.at[slot],
                t2_hbm.at[pl.ds(vb * (_VBLK // 2), _VBLK // 2)],
                out_sems[slot])

        def pair_body(p, _):
            for sl in (0, 1):
                i = p * 2 + sl

                @pl.when(i < n_mine)
                def _():
                    step(i, sl)
            return 0

        lax.fori_loop(0, (n_mine + 1) // 2, pair_body, 0)

        for sl in (0, 1):
            @pl.when(n_mine > sl)
            def _():
                pltpu.make_async_copy(
                    tout_v.at[sl],
                    t2_hbm.at[pl.ds(0, _VBLK // 2)],
                    out_sems[sl]).wait()

        @pl.when(wid == 0)
        def _():
            pltpu.sync_copy(tail2_hbm, tail_v_buf)
            pltpu.sync_copy(
                tail_v_buf,
                t2_hbm.at[pl.ds(n_blk * (_VBLK // 2), tail_v // 2)])

    return k


def _make_gather_kernel():
    info = plsc.get_sparse_core_info()
    nc, ns = info.num_cores, info.num_subcores
    nw = nc * ns
    n_tok = _BATCH * _LENGTH
    per_w = n_tok // nw
    n_ch = per_w // _CH
    ch_per_seq = _LENGTH // _CH
    mesh = plsc.VectorSubcoreMesh(core_axis_name="c", subcore_axis_name="s")

    @functools.partial(
        pl.kernel,
        out_type=jax.ShapeDtypeStruct((_BATCH, _DEPTH, _LENGTH), jnp.float32),
        mesh=mesh,
        scratch_types=[
            pltpu.VMEM((2, _CH), jnp.int32),
            pltpu.VMEM((2, _CH // 128, 128), jnp.int32),
            pltpu.VMEM((2, _CH), jnp.int32),
            pltpu.VMEM((_CH, 2 * _DEPTH), jnp.float32),
            pltpu.VMEM((_DEPTH, _CH), jnp.float32),
            pltpu.VMEM((_DEPTH * _LENGTH,), jnp.float32),
            pltpu.SemaphoreType.DMA,
            pltpu.SemaphoreType.DMA,
        ],
        compiler_params=pltpu.CompilerParams(use_tc_tiling_on_sc=True, needs_layout_passes=False, disable_bounds_checks=True),
    )
    def k(x_hbm, t2_hbm, pe_hbm, out_hbm, idx_v, q_v, pcol_v, wide_v,
          outb_v, pe_v, g_sem0, g_sem1):
        wid = lax.axis_index("s") * nc + lax.axis_index("c")
        base_w = wid * per_w
        pltpu.sync_copy(pe_hbm, pe_v)
        iota = lax.iota(jnp.int32, _LANES)
        g_sems = [g_sem0, g_sem1]

        def start_gather(s, slot):
            base = base_w + s * _CH
            pltpu.sync_copy(x_hbm.at[pl.ds(base, _CH)], idx_v.at[slot])

            def prep(i, _):
                v = idx_v[slot, pl.ds(i * _LANES, _LANES)]
                q_v[slot, i // 8, pl.ds((i % 8) * _LANES, _LANES)] = (
                    lax.shift_right_logical(v, 1))
                pcol_v[slot, pl.ds(i * _LANES, _LANES)] = lax.shift_left(
                    lax.bitwise_and(v, 1), 6)
                return 0

            lax.fori_loop(0, _CH // _LANES, prep, 0)
            for h in range(_CH // 128):
                pltpu.async_copy(
                    t2_hbm.at[q_v.at[slot, h]],
                    wide_v.at[pl.ds(h * 128, 128), :],
                    g_sems[slot])

        def step(s, slot):
            pltpu.make_async_copy(
                t2_hbm.at[pl.ds(0, _CH)], wide_v,
                g_sems[slot]).wait()

            base = base_w + s * _CH
            b = lax.div(base, _LENGTH)
            half = lax.rem(s, ch_per_seq)
            l_off = half * _CH

            def body_lb(lb, _):
                row16 = lb * _LANES + iota
                lb16 = lb * _LANES
                p_off = l_off + lb16
                p16 = pcol_v[slot, pl.ds(lb16, _LANES)]
                for d in range(_DEPTH):
                    g = plsc.load_gather(wide_v, [row16, p16 + d])
                    pe16 = pe_v[pl.ds(d * _LENGTH + p_off, _LANES)]
                    outb_v[d, pl.ds(lb16, _LANES)] = g + pe16
                return 0

            lax.fori_loop(0, _CH // _LANES, body_lb, 0)

            @pl.when(s + 1 < n_ch)
            def _():
                start_gather(s + 1, 1 - slot)

            pltpu.sync_copy(outb_v, out_hbm.at[b, :, pl.ds(l_off, _CH)])

        start_gather(0, 0)

        def pair_body(p, _):
            for sl in (0, 1):
                s = p * 2 + sl

                @pl.when(s < n_ch)
                def _():
                    step(s, sl)
            return 0

        lax.fori_loop(0, (n_ch + 1) // 2, pair_body, 0)

    return k


def kernel(x, table):
    pe_flat = jnp.asarray(_pos_encoding_t_np(_LENGTH, _DEPTH).reshape(-1))
    xf = x.reshape(-1).astype(jnp.int32)
    tt = table.T
    tail2 = table[_VOCAB - 64:].reshape(32, 128)
    k1 = _make_relayout_kernel()
    t2 = k1(tt, tail2)
    k2 = _make_gather_kernel()
    out_t = k2(xf, t2, pe_flat)
    return out_t.transpose(0, 2, 1)


# speedup vs baseline: 2.3223x; 2.3223x over previous
import functools

import jax
import jax.numpy as jnp
import numpy as np
from jax import lax
from jax.experimental import pallas as pl
from jax.experimental.pallas import tpu as pltpu
from jax.experimental.pallas import tpu_sc as plsc

_VOCAB = 1000000
_DEPTH = 64
_LENGTH = 512
_BATCH = 1024

_LANES = 16


def _pos_encoding_np(length, depth):
    pos = np.arange(length)[:, None]
    i = np.arange(depth)[None, :]
    angle_rates = 1.0 / np.power(10000, 2 * (i // 2) / np.float32(depth))
    angle_rads = pos * angle_rates
    angle_rads[:, 0::2] = np.sin(angle_rads[:, 0::2])
    angle_rads[:, 1::2] = np.cos(angle_rads[:, 1::2])
    return angle_rads.astype(np.float32)


def _make_sc_kernel(n_rows, depth, length):
    info = plsc.get_sparse_core_info()
    nc, ns = info.num_cores, info.num_subcores
    nw = nc * ns
    per_w = n_rows // nw
    ch = length
    n_ch = per_w // ch
    mesh = plsc.VectorSubcoreMesh(core_axis_name="c", subcore_axis_name="s")

    @functools.partial(
        pl.kernel,
        out_type=jax.ShapeDtypeStruct((n_rows, depth), jnp.float32),
        mesh=mesh,
        scratch_types=[
            pltpu.VMEM((ch,), jnp.int32),
            pltpu.VMEM((ch, depth), jnp.float32),
            pltpu.VMEM((ch, depth), jnp.float32),
            pltpu.SemaphoreType.DMA,
        ],
        compiler_params=pltpu.CompilerParams(use_tc_tiling_on_sc=False),
    )
    def k(x_hbm, table_hbm, pe_hbm, out_hbm, idx_v, rows_v, pe_v, sem):
        wid = lax.axis_index("s") * nc + lax.axis_index("c")
        base_w = wid * per_w
        pltpu.sync_copy(pe_hbm, pe_v)

        def chunk_body(s, carry):
            base = base_w + s * ch
            pltpu.sync_copy(x_hbm.at[pl.ds(base, ch)], idx_v)
            pltpu.async_copy(table_hbm.at[idx_v], rows_v, sem).wait()

            def row_body(r, c2):
                for c in range(depth // _LANES):
                    sl = pl.ds(c * _LANES, _LANES)
                    rows_v[r, sl] = rows_v[r, sl] + pe_v[r, sl]
                return c2

            lax.fori_loop(0, ch, row_body, 0)
            pltpu.sync_copy(rows_v, out_hbm.at[pl.ds(base, ch)])
            return carry

        lax.fori_loop(0, n_ch, chunk_body, 0)

    return k


def kernel(x, table):
    pe = jnp.asarray(_pos_encoding_np(_LENGTH, _DEPTH))
    xf = x.reshape(-1).astype(jnp.int32)
    k = _make_sc_kernel(xf.shape[0], _DEPTH, _LENGTH)
    out = k(xf, table, pe)
    return out.reshape(_BATCH, _LENGTH, _DEPTH)


# speedup vs baseline: 2.4552x; 1.0572x over previous
import functools

import jax
import jax.numpy as jnp
import numpy as np
from jax import lax
from jax.experimental import pallas as pl
from jax.experimental.pallas import tpu as pltpu
from jax.experimental.pallas import tpu_sc as plsc

_VOCAB = 1000000
_DEPTH = 64
_LENGTH = 512
_BATCH = 1024

_LANES = 16


def _pos_encoding_np(length, depth):
    pos = np.arange(length)[:, None]
    i = np.arange(depth)[None, :]
    angle_rates = 1.0 / np.power(10000, 2 * (i // 2) / np.float32(depth))
    angle_rads = pos * angle_rates
    angle_rads[:, 0::2] = np.sin(angle_rads[:, 0::2])
    angle_rads[:, 1::2] = np.cos(angle_rads[:, 1::2])
    return angle_rads.astype(np.float32)


def _make_sc_kernel(n_rows, depth, length):
    info = plsc.get_sparse_core_info()
    nc, ns = info.num_cores, info.num_subcores
    nw = nc * ns
    per_w = n_rows // nw
    ch = length
    n_ch = per_w // ch
    mesh = plsc.VectorSubcoreMesh(core_axis_name="c", subcore_axis_name="s")

    @functools.partial(
        pl.kernel,
        out_type=jax.ShapeDtypeStruct((n_rows, depth), jnp.float32),
        mesh=mesh,
        scratch_types=[
            pltpu.VMEM((2, ch), jnp.int32),
            pltpu.VMEM((2, ch, depth), jnp.float32),
            pltpu.VMEM((ch, depth), jnp.float32),
            pltpu.SemaphoreType.DMA,
            pltpu.SemaphoreType.DMA,
        ],
        compiler_params=pltpu.CompilerParams(use_tc_tiling_on_sc=False),
    )
    def k(x_hbm, table_hbm, pe_hbm, out_hbm, idx_v, rows_v, pe_v, sem0, sem1):
        wid = lax.axis_index("s") * nc + lax.axis_index("c")
        base_w = wid * per_w
        sems = [sem0, sem1]
        pltpu.sync_copy(pe_hbm, pe_v)

        def start_gather(s, slot):
            base = base_w + s * ch
            pltpu.sync_copy(x_hbm.at[pl.ds(base, ch)], idx_v.at[slot])
            pltpu.async_copy(table_hbm.at[idx_v.at[slot]], rows_v.at[slot],
                             sems[slot])

        def step(s, slot):
            @pl.when(s + 1 < n_ch)
            def _():
                start_gather(s + 1, 1 - slot)

            pltpu.make_async_copy(
                table_hbm.at[pl.ds(0, ch)], rows_v.at[slot],
                sems[slot]).wait()

            def row_body(r, c2):
                for c in range(depth // _LANES):
                    sl = pl.ds(c * _LANES, _LANES)
                    rows_v[slot, r, sl] = rows_v[slot, r, sl] + pe_v[r, sl]
                return c2

            lax.fori_loop(0, ch, row_body, 0)
            base = base_w + s * ch
            pltpu.sync_copy(rows_v.at[slot], out_hbm.at[pl.ds(base, ch)])

        start_gather(0, 0)

        def pair_body(p, _):
            for sl in (0, 1):
                s = p * 2 + sl

                @pl.when(s < n_ch)
                def _():
                    step(s, sl)
            return 0

        lax.fori_loop(0, (n_ch + 1) // 2, pair_body, 0)

    return k


def kernel(x, table):
    pe = jnp.asarray(_pos_encoding_np(_LENGTH, _DEPTH))
    xf = x.reshape(-1).astype(jnp.int32)
    k = _make_sc_kernel(xf.shape[0], _DEPTH, _LENGTH)
    out = k(xf, table, pe)
    return out.reshape(_BATCH, _LENGTH, _DEPTH)


# speedup vs baseline: 2.4628x; 1.0031x over previous
import functools

import jax
import jax.numpy as jnp
import numpy as np
from jax import lax
from jax.experimental import pallas as pl
from jax.experimental.pallas import tpu as pltpu
from jax.experimental.pallas import tpu_sc as plsc

_VOCAB = 1000000
_DEPTH = 64
_LENGTH = 512
_BATCH = 1024

_LANES = 16


def _pos_encoding_np(length, depth):
    pos = np.arange(length)[:, None]
    i = np.arange(depth)[None, :]
    angle_rates = 1.0 / np.power(10000, 2 * (i // 2) / np.float32(depth))
    angle_rads = pos * angle_rates
    angle_rads[:, 0::2] = np.sin(angle_rads[:, 0::2])
    angle_rads[:, 1::2] = np.cos(angle_rads[:, 1::2])
    return angle_rads.astype(np.float32)


def _make_sc_kernel(n_rows, depth, length):
    info = plsc.get_sparse_core_info()
    nc, ns = info.num_cores, info.num_subcores
    nw = nc * ns
    per_w = n_rows // nw
    ch = length
    n_ch = per_w // ch
    mesh = plsc.VectorSubcoreMesh(core_axis_name="c", subcore_axis_name="s")

    @functools.partial(
        pl.kernel,
        out_type=jax.ShapeDtypeStruct((n_rows, depth), jnp.float32),
        mesh=mesh,
        scratch_types=[
            pltpu.VMEM((2, ch), jnp.int32),
            pltpu.VMEM((2, ch, depth), jnp.float32),
            pltpu.VMEM((ch, depth), jnp.float32),
            pltpu.SemaphoreType.DMA,
            pltpu.SemaphoreType.DMA,
            pltpu.SemaphoreType.DMA,
            pltpu.SemaphoreType.DMA,
        ],
        compiler_params=pltpu.CompilerParams(use_tc_tiling_on_sc=False),
    )
    def k(x_hbm, table_hbm, pe_hbm, out_hbm, idx_v, rows_v, pe_v,
          sem0, sem1, osem0, osem1):
        wid = lax.axis_index("s") * nc + lax.axis_index("c")
        base_w = wid * per_w
        sems = [sem0, sem1]
        osems = [osem0, osem1]
        pltpu.sync_copy(pe_hbm, pe_v)

        def start_gather(s, slot):
            base = base_w + s * ch
            pltpu.sync_copy(x_hbm.at[pl.ds(base, ch)], idx_v.at[slot])
            pltpu.async_copy(table_hbm.at[idx_v.at[slot]], rows_v.at[slot],
                             sems[slot])

        def step(s, slot):
            @pl.when(s + 1 < n_ch)
            def _():
                @pl.when(s >= 1)
                def _():
                    pltpu.make_async_copy(
                        rows_v.at[1 - slot],
                        out_hbm.at[pl.ds(0, ch)],
                        osems[1 - slot]).wait()

                start_gather(s + 1, 1 - slot)

            pltpu.make_async_copy(
                table_hbm.at[pl.ds(0, ch)], rows_v.at[slot],
                sems[slot]).wait()

            def row_body(r, c2):
                for c in range(depth // _LANES):
                    sl = pl.ds(c * _LANES, _LANES)
                    rows_v[slot, r, sl] = rows_v[slot, r, sl] + pe_v[r, sl]
                return c2

            lax.fori_loop(0, ch, row_body, 0)
            base = base_w + s * ch
            pltpu.async_copy(rows_v.at[slot], out_hbm.at[pl.ds(base, ch)],
                             osems[slot])

        start_gather(0, 0)

        def pair_body(p, _):
            for sl in (0, 1):
                s = p * 2 + sl

                @pl.when(s < n_ch)
                def _():
                    step(s, sl)
            return 0

        lax.fori_loop(0, (n_ch + 1) // 2, pair_body, 0)

        for sl in (0, 1):
            @pl.when(n_ch > sl)
            def _():
                pltpu.make_async_copy(
                    rows_v.at[sl], out_hbm.at[pl.ds(0, ch)],
                    osems[sl]).wait()

    return k


def kernel(x, table):
    pe = jnp.asarray(_pos_encoding_np(_LENGTH, _DEPTH))
    xf = x.reshape(-1).astype(jnp.int32)
    k = _make_sc_kernel(xf.shape[0], _DEPTH, _LENGTH)
    out = k(xf, table, pe)
    return out.reshape(_BATCH, _LENGTH, _DEPTH)
